# baseline (speedup 1.0000x reference)
import numpy as np
import jax
import jax.numpy as jnp
from jax.experimental import pallas as pl

_ISIZE = 224
_PSIZE = 30
_rng = np.random.RandomState(0)
_X = int(_rng.randint(0, _ISIZE - _PSIZE))
_Y = int(_rng.randint(0, _ISIZE - _PSIZE))


def _patch_add_kernel(x_ref, p_ref, o_ref):
    o_ref[...] = x_ref[...]
    o_ref[:, :, _X:_X + _PSIZE, _Y:_Y + _PSIZE] = (
        x_ref[:, :, _X:_X + _PSIZE, _Y:_Y + _PSIZE] + p_ref[...]
    )


def kernel(x, patch):
    B = x.shape[0]
    Bb = 4
    return pl.pallas_call(
        _patch_add_kernel,
        grid=(B // Bb,),
        in_specs=[
            pl.BlockSpec((Bb, 3, _ISIZE, _ISIZE), lambda i: (i, 0, 0, 0)),
            pl.BlockSpec((1, 3, _PSIZE, _PSIZE), lambda i: (0, 0, 0, 0)),
        ],
        out_specs=pl.BlockSpec((Bb, 3, _ISIZE, _ISIZE), lambda i: (i, 0, 0, 0)),
        out_shape=jax.ShapeDtypeStruct(x.shape, x.dtype),
    )(x, patch)


# speedup vs baseline: 1.0041x; 1.0041x over previous
import numpy as np
import jax
import jax.numpy as jnp
from jax.experimental import pallas as pl

_ISIZE = 224
_PSIZE = 30
_rng = np.random.RandomState(0)
_X = int(_rng.randint(0, _ISIZE - _PSIZE))
_Y = int(_rng.randint(0, _ISIZE - _PSIZE))


def _canvas_kernel(p_ref, c_ref):
    c_ref[...] = jnp.zeros_like(c_ref)
    c_ref[:, :, _X:_X + _PSIZE, _Y:_Y + _PSIZE] = p_ref[...]


def _add_kernel(x_ref, c_ref, o_ref):
    o_ref[...] = x_ref[...] + c_ref[...]


def kernel(x, patch):
    B = x.shape[0]
    canvas = pl.pallas_call(
        _canvas_kernel,
        out_shape=jax.ShapeDtypeStruct((1, 3, _ISIZE, _ISIZE), x.dtype),
    )(patch)
    Bb = 8
    return pl.pallas_call(
        _add_kernel,
        grid=(B // Bb,),
        in_specs=[
            pl.BlockSpec((Bb, 3, _ISIZE, _ISIZE), lambda i: (i, 0, 0, 0)),
            pl.BlockSpec((1, 3, _ISIZE, _ISIZE), lambda i: (0, 0, 0, 0)),
        ],
        out_specs=pl.BlockSpec((Bb, 3, _ISIZE, _ISIZE), lambda i: (i, 0, 0, 0)),
        out_shape=jax.ShapeDtypeStruct(x.shape, x.dtype),
    )(x, canvas)


# speedup vs baseline: 1.1365x; 1.1318x over previous
import numpy as np
import jax
import jax.numpy as jnp
from jax.experimental import pallas as pl

_ISIZE = 224
_PSIZE = 30
_rng = np.random.RandomState(0)
_X = int(_rng.randint(0, _ISIZE - _PSIZE))
_Y = int(_rng.randint(0, _ISIZE - _PSIZE))


def _canvas_kernel(p_ref, c_ref):
    c_ref[...] = jnp.zeros_like(c_ref)
    c_ref[:, :, _X:_X + _PSIZE, _Y:_Y + _PSIZE] = p_ref[...]


def _add_kernel(x_ref, c_ref, o_ref):
    o_ref[...] = x_ref[...] + c_ref[...]


def kernel(x, patch):
    B = x.shape[0]
    canvas = pl.pallas_call(
        _canvas_kernel,
        out_shape=jax.ShapeDtypeStruct((1, 3, _ISIZE, _ISIZE), x.dtype),
    )(patch)
    flat = 3 * _ISIZE * _ISIZE
    rows = flat // 128
    x2 = x.reshape(B, rows, 128)
    c2 = canvas.reshape(1, rows, 128)
    Bb = 8
    out = pl.pallas_call(
        _add_kernel,
        grid=(B // Bb,),
        in_specs=[
            pl.BlockSpec((Bb, rows, 128), lambda i: (i, 0, 0)),
            pl.BlockSpec((1, rows, 128), lambda i: (0, 0, 0)),
        ],
        out_specs=pl.BlockSpec((Bb, rows, 128), lambda i: (i, 0, 0)),
        out_shape=jax.ShapeDtypeStruct((B, rows, 128), x.dtype),
    )(x2, c2)
    return out.reshape(x.shape)
